# baseline (speedup 1.0000x reference)
import jax
import jax.numpy as jnp
from jax.experimental import pallas as pl
from jax.experimental.pallas import tpu as pltpu

_N, _DIN, _H1, _H2 = 4096, 128, 64, 32
_BA = 512
_NA = _N // _BA
_BB = 512
_BD = 1024
_ND = _N // _BD


def _enc_body(adj_hbm, x_ref, w1_ref, wc_ref,
              mlv_ref,
              adjb, s1, hw, buf0, buf1, sem0, sem1):

    def cp_in(i, buf, sem):
        return pltpu.make_async_copy(
            adj_hbm.at[pl.ds(i * _BA, _BA), :], buf, sem)

    s1[...] = jnp.dot(
        x_ref[...], w1_ref[...], preferred_element_type=jnp.float32
    ).astype(jnp.bfloat16)

    cp_in(0, buf0, sem0).start()

    def step_a(s, carry):
        def work(buf, sem, obuf, osem):
            cp_in(s, buf, sem).wait()

            @pl.when(s + 1 < _NA)
            def _():
                cp_in(s + 1, obuf, osem).start()
            a = buf[...].astype(jnp.bfloat16)
            adjb[pl.ds(s * _BA, _BA), :] = a
            h = jnp.dot(a, s1[...], preferred_element_type=jnp.float32)
            h = jnp.maximum(h, 0.0).astype(jnp.bfloat16)
            hw[pl.ds(s * _BA, _BA), :] = jnp.dot(
                h, wc_ref[...], preferred_element_type=jnp.float32
            ).astype(jnp.bfloat16)

        @pl.when(s % 2 == 0)
        def _even():
            work(buf0, sem0, buf1, sem1)

        @pl.when(s % 2 == 1)
        def _odd():
            work(buf1, sem1, buf0, sem0)

        return carry

    jax.lax.fori_loop(0, _NA, step_a, 0)

    def step_b(m, carry):
        a = adjb[pl.ds(m * _BB, _BB), :]
        mlv_ref[pl.ds(m * _BB, _BB), :] = jnp.dot(
            a, hw[...], preferred_element_type=jnp.float32)
        return carry

    jax.lax.fori_loop(0, _N // _BB, step_b, 0)


def _dec_body(zi_ref, z_ref, o_ref):
    zz = jax.lax.dot_general(
        zi_ref[...], z_ref[...], (((1,), (1,)), ((), ())),
        preferred_element_type=jnp.float32,
    )
    o_ref[...] = jax.nn.sigmoid(zz)


def kernel(x, adj, W1, W2, W3):
    wc = jnp.concatenate([W2, W3], axis=1).astype(jnp.bfloat16)

    mlv = pl.pallas_call(
        _enc_body,
        in_specs=[
            pl.BlockSpec(memory_space=pl.ANY),
            pl.BlockSpec(memory_space=pltpu.MemorySpace.VMEM),
            pl.BlockSpec(memory_space=pltpu.MemorySpace.VMEM),
            pl.BlockSpec(memory_space=pltpu.MemorySpace.VMEM),
        ],
        out_specs=pl.BlockSpec(memory_space=pltpu.MemorySpace.VMEM),
        out_shape=jax.ShapeDtypeStruct((_N, 2 * _H2), jnp.float32),
        scratch_shapes=[
            pltpu.VMEM((_N, _N), jnp.bfloat16),
            pltpu.VMEM((_N, _H1), jnp.bfloat16),
            pltpu.VMEM((_N, 2 * _H2), jnp.bfloat16),
            pltpu.VMEM((_BA, _N), jnp.float32),
            pltpu.VMEM((_BA, _N), jnp.float32),
            pltpu.SemaphoreType.DMA,
            pltpu.SemaphoreType.DMA,
        ],
    )(adj, x, W1, wc)

    mu = mlv[:, :_H2]
    logvar = mlv[:, _H2:]
    zb = mu.astype(jnp.bfloat16)

    decode = pl.pallas_call(
        _dec_body,
        grid=(_ND,),
        in_specs=[
            pl.BlockSpec((_BD, _H2), lambda i: (i, 0)),
            pl.BlockSpec((_N, _H2), lambda i: (0, 0)),
        ],
        out_specs=pl.BlockSpec((_BD, _N), lambda i: (i, 0)),
        out_shape=jax.ShapeDtypeStruct((_N, _N), jnp.float32),
    )(zb, zb)

    return decode, mu, logvar


# speedup vs baseline: 1.0846x; 1.0846x over previous
import jax
import jax.numpy as jnp
from jax.experimental import pallas as pl
from jax.experimental.pallas import tpu as pltpu

_N, _DIN, _H1, _H2 = 4096, 128, 64, 32
_BA = 512
_NA = _N // _BA
_BB = 512
_BD = 512
_ND = _N // _BD


def _enc_body(adj_ref, x_ref, w1_ref, wc_ref, mlv_ref, adjb, s1, hw):
    s = pl.program_id(0)

    @pl.when(s == 0)
    def _init_s1():
        s1[...] = jnp.dot(
            x_ref[...], w1_ref[...], preferred_element_type=jnp.float32
        ).astype(jnp.bfloat16)

    @pl.when(s < _NA)
    def _phase_a():
        a = adj_ref[...].astype(jnp.bfloat16)
        adjb[pl.ds(s * _BA, _BA), :] = a
        h = jnp.dot(a, s1[...], preferred_element_type=jnp.float32)
        h = jnp.maximum(h, 0.0).astype(jnp.bfloat16)
        hw[pl.ds(s * _BA, _BA), :] = jnp.dot(
            h, wc_ref[...], preferred_element_type=jnp.float32
        ).astype(jnp.bfloat16)

    @pl.when(s == _NA)
    def _phase_b():
        def body(m, _):
            a = adjb[pl.ds(m * _BB, _BB), :]
            mlv_ref[pl.ds(m * _BB, _BB), :] = jnp.dot(
                a, hw[...], preferred_element_type=jnp.float32)
            return 0
        jax.lax.fori_loop(0, _N // _BB, body, 0)


def _dec_body(z_ref, dec_hbm, buf0, buf1, sem0, sem1):

    def cp_out(i, buf, sem):
        return pltpu.make_async_copy(
            buf, dec_hbm.at[pl.ds(i * _BD, _BD), :], sem)

    def step_c(i, carry):
        def work(buf, sem):
            @pl.when(i >= 2)
            def _():
                cp_out(i - 2, buf, sem).wait()
            zi = z_ref[pl.ds(i * _BD, _BD), :]
            zz = jax.lax.dot_general(
                zi, z_ref[...], (((1,), (1,)), ((), ())),
                preferred_element_type=jnp.float32,
            )
            buf[...] = jax.nn.sigmoid(zz)
            cp_out(i, buf, sem).start()

        @pl.when(i % 2 == 0)
        def _even():
            work(buf0, sem0)

        @pl.when(i % 2 == 1)
        def _odd():
            work(buf1, sem1)

        return carry

    jax.lax.fori_loop(0, _ND, step_c, 0)
    cp_out(_ND - 2, buf0, sem0).wait()
    cp_out(_ND - 1, buf1, sem1).wait()


def kernel(x, adj, W1, W2, W3):
    wc = jnp.concatenate([W2, W3], axis=1).astype(jnp.bfloat16)

    mlv = pl.pallas_call(
        _enc_body,
        grid=(_NA + 1,),
        in_specs=[
            pl.BlockSpec((_BA, _N), lambda s: (jnp.minimum(s, _NA - 1), 0)),
            pl.BlockSpec((_N, _DIN), lambda s: (0, 0)),
            pl.BlockSpec((_DIN, _H1), lambda s: (0, 0)),
            pl.BlockSpec((_H1, 2 * _H2), lambda s: (0, 0)),
        ],
        out_specs=pl.BlockSpec((_N, 2 * _H2), lambda s: (0, 0)),
        out_shape=jax.ShapeDtypeStruct((_N, 2 * _H2), jnp.float32),
        scratch_shapes=[
            pltpu.VMEM((_N, _N), jnp.bfloat16),
            pltpu.VMEM((_N, _H1), jnp.bfloat16),
            pltpu.VMEM((_N, 2 * _H2), jnp.bfloat16),
        ],
    )(adj, x, W1, wc)

    mu = mlv[:, :_H2]
    logvar = mlv[:, _H2:]
    zb = mu.astype(jnp.bfloat16)

    decode = pl.pallas_call(
        _dec_body,
        in_specs=[pl.BlockSpec(memory_space=pltpu.MemorySpace.VMEM)],
        out_specs=pl.BlockSpec(memory_space=pl.ANY),
        out_shape=jax.ShapeDtypeStruct((_N, _N), jnp.float32),
        scratch_shapes=[
            pltpu.VMEM((_BD, _N), jnp.float32),
            pltpu.VMEM((_BD, _N), jnp.float32),
            pltpu.SemaphoreType.DMA,
            pltpu.SemaphoreType.DMA,
        ],
    )(zb)

    return decode, mu, logvar


# speedup vs baseline: 1.1386x; 1.0497x over previous
import jax
import jax.numpy as jnp
from jax.experimental import pallas as pl
from jax.experimental.pallas import tpu as pltpu

_N, _DIN, _H1, _H2 = 4096, 128, 64, 32
_BA = 512
_NA = _N // _BA
_BB = 512
_BD = 512
_ND = _N // _BD


def _enc_body(adj_ref, x_ref, w1_ref, wc_ref, mlv_ref, adjb, s1, hw):
    s = pl.program_id(0)

    @pl.when(s == 0)
    def _init_s1():
        s1[...] = jnp.dot(
            x_ref[...], w1_ref[...], preferred_element_type=jnp.float32
        ).astype(jnp.bfloat16)

    @pl.when(s < _NA)
    def _phase_a():
        a = adj_ref[...].astype(jnp.bfloat16)
        adjb[pl.ds(s * _BA, _BA), :] = a
        h = jnp.dot(a, s1[...], preferred_element_type=jnp.float32)
        h = jnp.maximum(h, 0.0).astype(jnp.bfloat16)
        hw[pl.ds(s * _BA, _BA), :] = jnp.dot(
            h, wc_ref[...], preferred_element_type=jnp.float32
        ).astype(jnp.bfloat16)

    @pl.when(s == _NA)
    def _phase_b():
        def body(m, _):
            a = adjb[pl.ds(m * _BB, _BB), :]
            mlv_ref[pl.ds(m * _BB, _BB), :] = jnp.dot(
                a, hw[...], preferred_element_type=jnp.float32)
            return 0
        jax.lax.fori_loop(0, _N // _BB, body, 0)


def _dec_body(z_ref, dec_hbm, buf0, buf1, sem0, sem1):

    def cp_out(i, buf, sem):
        return pltpu.make_async_copy(
            buf, dec_hbm.at[pl.ds(i * _BD, _BD), :], sem)

    def step_c(i, carry):
        def work(buf, sem):
            @pl.when(i >= 2)
            def _():
                cp_out(i - 2, buf, sem).wait()
            zi = z_ref[pl.ds(i * _BD, _BD), :]
            zz = jax.lax.dot_general(
                zi, z_ref[...], (((1,), (1,)), ((), ())),
                preferred_element_type=jnp.float32,
            )
            buf[...] = 0.5 * jnp.tanh(0.5 * zz) + 0.5
            cp_out(i, buf, sem).start()

        @pl.when(i % 2 == 0)
        def _even():
            work(buf0, sem0)

        @pl.when(i % 2 == 1)
        def _odd():
            work(buf1, sem1)

        return carry

    jax.lax.fori_loop(0, _ND, step_c, 0)
    cp_out(_ND - 2, buf0, sem0).wait()
    cp_out(_ND - 1, buf1, sem1).wait()


def kernel(x, adj, W1, W2, W3):
    wc = jnp.concatenate([W2, W3], axis=1).astype(jnp.bfloat16)

    mlv = pl.pallas_call(
        _enc_body,
        grid=(_NA + 1,),
        in_specs=[
            pl.BlockSpec((_BA, _N), lambda s: (jnp.minimum(s, _NA - 1), 0)),
            pl.BlockSpec((_N, _DIN), lambda s: (0, 0)),
            pl.BlockSpec((_DIN, _H1), lambda s: (0, 0)),
            pl.BlockSpec((_H1, 2 * _H2), lambda s: (0, 0)),
        ],
        out_specs=pl.BlockSpec((_N, 2 * _H2), lambda s: (0, 0)),
        out_shape=jax.ShapeDtypeStruct((_N, 2 * _H2), jnp.float32),
        scratch_shapes=[
            pltpu.VMEM((_N, _N), jnp.bfloat16),
            pltpu.VMEM((_N, _H1), jnp.bfloat16),
            pltpu.VMEM((_N, 2 * _H2), jnp.bfloat16),
        ],
    )(adj, x, W1, wc)

    mu = mlv[:, :_H2]
    logvar = mlv[:, _H2:]
    zb = mu.astype(jnp.bfloat16)

    decode = pl.pallas_call(
        _dec_body,
        in_specs=[pl.BlockSpec(memory_space=pltpu.MemorySpace.VMEM)],
        out_specs=pl.BlockSpec(memory_space=pl.ANY),
        out_shape=jax.ShapeDtypeStruct((_N, _N), jnp.float32),
        scratch_shapes=[
            pltpu.VMEM((_BD, _N), jnp.float32),
            pltpu.VMEM((_BD, _N), jnp.float32),
            pltpu.SemaphoreType.DMA,
            pltpu.SemaphoreType.DMA,
        ],
    )(zb)

    return decode, mu, logvar
